# baseline (speedup 1.0000x reference)
import dataclasses
import functools

import jax
import jax.numpy as jnp
from jax import lax
from jax.experimental import pallas as pl
from jax.experimental.pallas import tpu as pltpu
from jax.experimental.pallas import tpu_sc as plsc

_N = 10000
_NPAD = 10240
_D = 128
_OUT = 64
_ROW = 144
_E = 320000
_NC = 2
_NS = 16
_NW = _NC * _NS
_CH = 128
_NCHUNK = 81
_SEG = 41
_EPAD = _CH * _NW * _NCHUNK
_ACC = 10016
_RPT = _ACC // _NS
_BN = 1024


def _aug_cols(a):
    one = jnp.ones((_BN, 1), jnp.float32)
    zero = jnp.zeros((_BN, _ROW - _D - 2), jnp.float32)
    return jnp.concatenate([one, a[:, 0:1], zero], axis=1)


def _embed_body(x_ref, w_ref, am_ref, haug_ref, av_ref):
    h = jnp.dot(x_ref[...], w_ref[...], preferred_element_type=jnp.float32)
    a = jnp.dot(h, am_ref[...], preferred_element_type=jnp.float32)
    av_ref[...] = jnp.concatenate(
        [a[:, 1:2].T, jnp.zeros((7, _BN), jnp.float32)], axis=0)
    haug_ref[...] = jnp.concatenate([h, _aug_cols(a)], axis=1)


def _embed(xp, w, am):
    return pl.pallas_call(
        _embed_body,
        grid=(_NPAD // _BN,),
        in_specs=[
            pl.BlockSpec((_BN, _D), lambda i: (i, 0)),
            pl.BlockSpec((_D, _D), lambda i: (0, 0)),
            pl.BlockSpec((_D, 8), lambda i: (0, 0)),
        ],
        out_specs=[
            pl.BlockSpec((_BN, _ROW), lambda i: (i, 0)),
            pl.BlockSpec((8, _BN), lambda i: (0, i)),
        ],
        out_shape=[
            jax.ShapeDtypeStruct((_NPAD, _ROW), jnp.float32),
            jax.ShapeDtypeStruct((8, _NPAD), jnp.float32),
        ],
    )(xp, w, am)


def _combine_body(p0_ref, p1_ref, hg_ref, am1_ref, b1_ref, w2_ref, am2_ref,
                  haug_ref, av_ref):
    hg = hg_ref[...]
    h = hg[:, :_D]
    a = jnp.dot(h, am1_ref[...], preferred_element_type=jnp.float32)
    es = a[:, 0:1] + a[:, 1:2]
    es = jnp.where(es >= 0.0, es, 0.2 * es)
    tot = p0_ref[0] + p1_ref[0] + jnp.exp(es) * hg
    hact = jnp.maximum(tot[:, :_D] / tot[:, _D:_D + 1] + b1_ref[...], 0.0)
    h2 = jnp.dot(hact, w2_ref[...], preferred_element_type=jnp.float32)
    a2 = jnp.dot(h2, am2_ref[...], preferred_element_type=jnp.float32)
    av_ref[...] = jnp.concatenate(
        [a2[:, 1:2].T, jnp.zeros((7, _BN), jnp.float32)], axis=0)
    haug_ref[...] = jnp.concatenate([h2, _aug_cols(a2)], axis=1)


def _combine(parts, haug1, am1, b1, w2, am2):
    return pl.pallas_call(
        _combine_body,
        grid=(_NPAD // _BN,),
        in_specs=[
            pl.BlockSpec((1, _BN, _ROW), lambda i: (0, i, 0)),
            pl.BlockSpec((1, _BN, _ROW), lambda i: (1, i, 0)),
            pl.BlockSpec((_BN, _ROW), lambda i: (i, 0)),
            pl.BlockSpec((_D, 8), lambda i: (0, 0)),
            pl.BlockSpec((1, _D), lambda i: (0, 0)),
            pl.BlockSpec((_D, _D), lambda i: (0, 0)),
            pl.BlockSpec((_D, 8), lambda i: (0, 0)),
        ],
        out_specs=[
            pl.BlockSpec((_BN, _ROW), lambda i: (i, 0)),
            pl.BlockSpec((8, _BN), lambda i: (0, i)),
        ],
        out_shape=[
            jax.ShapeDtypeStruct((_NPAD, _ROW), jnp.float32),
            jax.ShapeDtypeStruct((8, _NPAD), jnp.float32),
        ],
    )(parts, parts, haug1, am1, b1, w2, am2)


def _final_body(p0_ref, p1_ref, hg_ref, am2_ref, b2_ref, w1_ref, wb1_ref,
                w2_ref, wb2_ref, out_ref):
    hg = hg_ref[...]
    h = hg[:, :_D]
    a = jnp.dot(h, am2_ref[...], preferred_element_type=jnp.float32)
    es = a[:, 0:1] + a[:, 1:2]
    es = jnp.where(es >= 0.0, es, 0.2 * es)
    tot = p0_ref[0] + p1_ref[0] + jnp.exp(es) * hg
    g = jnp.maximum(tot[:, :_D] / tot[:, _D:_D + 1] + b2_ref[...], 0.0)
    z = jnp.maximum(
        jnp.dot(g, w1_ref[...], preferred_element_type=jnp.float32)
        + wb1_ref[...], 0.0)
    logits = jnp.dot(z, w2_ref[...], preferred_element_type=jnp.float32)
    logits = logits + wb2_ref[...]
    m = jnp.max(logits, axis=1, keepdims=True)
    p = jnp.exp(logits - m)
    out_ref[...] = p / jnp.sum(p, axis=1, keepdims=True)


def _final(parts, haug2, am2, b2, fc1_w, fc1_b, fc2_w, fc2_b):
    return pl.pallas_call(
        _final_body,
        grid=(_NPAD // _BN,),
        in_specs=[
            pl.BlockSpec((1, _BN, _ROW), lambda i: (0, i, 0)),
            pl.BlockSpec((1, _BN, _ROW), lambda i: (1, i, 0)),
            pl.BlockSpec((_BN, _ROW), lambda i: (i, 0)),
            pl.BlockSpec((_D, 8), lambda i: (0, 0)),
            pl.BlockSpec((1, _D), lambda i: (0, 0)),
            pl.BlockSpec((_D, _D), lambda i: (0, 0)),
            pl.BlockSpec((1, _D), lambda i: (0, 0)),
            pl.BlockSpec((_D, _OUT), lambda i: (0, 0)),
            pl.BlockSpec((1, _OUT), lambda i: (0, 0)),
        ],
        out_specs=pl.BlockSpec((_BN, _OUT), lambda i: (i, 0)),
        out_shape=jax.ShapeDtypeStruct((_NPAD, _OUT), jnp.float32),
    )(parts, parts, haug2, am2, b2, fc1_w, fc1_b, fc2_w, fc2_b)


def _edge_body(src_hbm, dst_hbm, av_hbm, haug_hbm, out_hbm,
               slab_s, slab_d, srci, dsti, eebuf, adst_v, rows, accum):
    c = lax.axis_index("c")
    s = lax.axis_index("s")
    wid = c * _NS + s
    wbase = wid * _NCHUNK

    @pl.loop(0, _CH)
    def _zero(r):
        for k in range(_ROW // 16):
            rows[r, pl.ds(k * 16, 16)] = jnp.zeros((16,), jnp.float32)

    for t in range(_RPT // _CH):
        pltpu.sync_copy(rows, accum.at[pl.ds(s * _RPT + t * _CH, _CH)])
    rem = _RPT % _CH
    if rem:
        pltpu.sync_copy(rows.at[pl.ds(0, rem)],
                        accum.at[pl.ds(s * _RPT + _RPT - rem, rem)])
    pltpu.sync_copy(av_hbm.at[0], adst_v)
    plsc.subcore_barrier()

    def chunk_body(row):
        @pl.loop(0, _CH, step=16)
        def _stage(j):
            srci[pl.ds(j, 16)] = slab_s[row, pl.ds(j, 16)]
            dsti[pl.ds(j, 16)] = slab_d[row, pl.ds(j, 16)]

        pltpu.sync_copy(haug_hbm.at[srci], rows)
        lane = jnp.arange(16, dtype=jnp.int32)

        @pl.loop(0, _CH, step=16)
        def _e(j):
            ridx = j + lane
            a_s = plsc.load_gather(rows, [ridx,
                                          jnp.full((16,), 129, jnp.int32)])
            a_d = plsc.load_gather(adst_v, [dsti[pl.ds(j, 16)]])
            e = a_s + a_d
            e = jnp.where(e >= 0.0, e, 0.2 * e)
            eebuf[pl.ds(j, 16)] = jnp.exp(e)

        @pl.loop(0, _CH, step=16)
        def _w(j):
            ee = eebuf[pl.ds(j, 16)]
            for t in range(16):
                v = jnp.full((16,), ee[t], jnp.float32)
                for k in range(_ROW // 16):
                    rows[j + t, pl.ds(k * 16, 16)] = (
                        rows[j + t, pl.ds(k * 16, 16)] * v)

        pltpu.sync_copy(rows, accum.at[dsti], add=True)

    pltpu.sync_copy(src_hbm.at[pl.ds(wbase, _SEG)], slab_s)
    pltpu.sync_copy(dst_hbm.at[pl.ds(wbase, _SEG)], slab_d)

    @pl.loop(0, _SEG)
    def _loop1(g):
        chunk_body(g)

    seg2 = _NCHUNK - _SEG
    pltpu.sync_copy(src_hbm.at[pl.ds(wbase + _SEG, seg2)],
                    slab_s.at[pl.ds(0, seg2)])
    pltpu.sync_copy(dst_hbm.at[pl.ds(wbase + _SEG, seg2)],
                    slab_d.at[pl.ds(0, seg2)])

    @pl.loop(0, seg2)
    def _loop2(g):
        chunk_body(g)

    plsc.subcore_barrier()
    pltpu.sync_copy(accum.at[pl.ds(s * _RPT, _RPT)],
                    out_hbm.at[c, pl.ds(s * _RPT, _RPT)])


def _edge_pass(src, dst, av, haug):
    mesh = plsc.VectorSubcoreMesh(core_axis_name="c", subcore_axis_name="s",
                                  num_cores=_NC, num_subcores=_NS)
    cp = pltpu.CompilerParams()
    if "needs_layout_passes" in pltpu.CompilerParams.__dataclass_fields__:
        cp = dataclasses.replace(cp, needs_layout_passes=False,
                                 use_tc_tiling_on_sc=False)
    f = pl.kernel(
        _edge_body,
        out_type=jax.ShapeDtypeStruct((_NC, _NPAD, _ROW), jnp.float32),
        mesh=mesh,
        scratch_types=[
            pltpu.VMEM((_SEG, _CH), jnp.int32),
            pltpu.VMEM((_SEG, _CH), jnp.int32),
            pltpu.VMEM((_CH,), jnp.int32),
            pltpu.VMEM((_CH,), jnp.int32),
            pltpu.VMEM((_CH,), jnp.float32),
            pltpu.VMEM((_NPAD,), jnp.float32),
            pltpu.VMEM((_CH, _ROW), jnp.float32),
            pltpu.VMEM_SHARED((_ACC, _ROW), jnp.float32),
        ],
        compiler_params=cp,
    )
    return f(src, dst, av, haug)


def kernel(x, edge_index, W1, att_s1, att_d1, b1, W2, att_s2, att_d2, b2,
           fc1_w, fc1_b, fc2_w, fc2_b):
    xp = jnp.zeros((_NPAD, _D), jnp.float32).at[:_N].set(x)
    src = (jnp.full((_EPAD,), _N, jnp.int32).at[:_E].set(edge_index[0])
           .reshape(_EPAD // _CH, _CH))
    dst = (jnp.full((_EPAD,), _N, jnp.int32).at[:_E].set(edge_index[1])
           .reshape(_EPAD // _CH, _CH))
    am1 = (jnp.zeros((_D, 8), jnp.float32)
           .at[:, 0].set(att_s1[0]).at[:, 1].set(att_d1[0]))
    am2 = (jnp.zeros((_D, 8), jnp.float32)
           .at[:, 0].set(att_s2[0]).at[:, 1].set(att_d2[0]))

    haug1, av1 = _embed(xp, W1, am1)
    parts1 = _edge_pass(src, dst, av1, haug1)
    haug2, av2 = _combine(parts1, haug1, am1, b1.reshape(1, _D), W2, am2)
    parts2 = _edge_pass(src, dst, av2, haug2)
    out = _final(parts2, haug2, am2, b2.reshape(1, _D),
                 fc1_w, fc1_b.reshape(1, _D), fc2_w, fc2_b.reshape(1, _OUT))
    return out[:_N]


# speedup vs baseline: 1.4950x; 1.4950x over previous
import dataclasses
import functools

import jax
import jax.numpy as jnp
from jax import lax
from jax.experimental import pallas as pl
from jax.experimental.pallas import tpu as pltpu
from jax.experimental.pallas import tpu_sc as plsc

_N = 10000
_NPAD = 10240
_D = 128
_OUT = 64
_ROW = 144
_E = 320000
_NC = 2
_NS = 16
_NW = _NC * _NS
_CH = 128
_NCHUNK = 79
_EPW = _CH * _NCHUNK
_EPAD = _EPW * _NW
_ACC = 10176
_RPT = _ACC // _NS
_BN = 1024


def _embed_body(x_ref, w_ref, am_ref, haug_ref, av_ref):
    h = jnp.dot(x_ref[...], w_ref[...], preferred_element_type=jnp.float32)
    a = jnp.dot(h, am_ref[...], preferred_element_type=jnp.float32)
    av_ref[...] = a.T
    ex = jnp.where(
        lax.broadcasted_iota(jnp.int32, (_BN, _ROW - _D), 1) == 0, 1.0, 0.0)
    haug_ref[...] = jnp.concatenate([h, ex], axis=1)


def _embed(xp, w, am):
    return pl.pallas_call(
        _embed_body,
        grid=(_NPAD // _BN,),
        in_specs=[
            pl.BlockSpec((_BN, _D), lambda i: (i, 0)),
            pl.BlockSpec((_D, _D), lambda i: (0, 0)),
            pl.BlockSpec((_D, 8), lambda i: (0, 0)),
        ],
        out_specs=[
            pl.BlockSpec((_BN, _ROW), lambda i: (i, 0)),
            pl.BlockSpec((8, _BN), lambda i: (0, i)),
        ],
        out_shape=[
            jax.ShapeDtypeStruct((_NPAD, _ROW), jnp.float32),
            jax.ShapeDtypeStruct((8, _NPAD), jnp.float32),
        ],
    )(xp, w, am)


def _combine_body(p0_ref, p1_ref, hg_ref, am1_ref, b1_ref, w2_ref, am2_ref,
                  haug_ref, av_ref):
    hg = hg_ref[...]
    h = hg[:, :_D]
    a = jnp.dot(h, am1_ref[...], preferred_element_type=jnp.float32)
    es = a[:, 0:1] + a[:, 1:2]
    es = jnp.where(es >= 0.0, es, 0.2 * es)
    tot = p0_ref[0] + p1_ref[0] + jnp.exp(es) * hg
    hact = jnp.maximum(tot[:, :_D] / tot[:, _D:_D + 1] + b1_ref[...], 0.0)
    h2 = jnp.dot(hact, w2_ref[...], preferred_element_type=jnp.float32)
    a2 = jnp.dot(h2, am2_ref[...], preferred_element_type=jnp.float32)
    av_ref[...] = a2.T
    ex = jnp.where(
        lax.broadcasted_iota(jnp.int32, (_BN, _ROW - _D), 1) == 0, 1.0, 0.0)
    haug_ref[...] = jnp.concatenate([h2, ex], axis=1)


def _combine(parts, haug1, am1, b1, w2, am2):
    return pl.pallas_call(
        _combine_body,
        grid=(_NPAD // _BN,),
        in_specs=[
            pl.BlockSpec((1, _BN, _ROW), lambda i: (0, i, 0)),
            pl.BlockSpec((1, _BN, _ROW), lambda i: (1, i, 0)),
            pl.BlockSpec((_BN, _ROW), lambda i: (i, 0)),
            pl.BlockSpec((_D, 8), lambda i: (0, 0)),
            pl.BlockSpec((1, _D), lambda i: (0, 0)),
            pl.BlockSpec((_D, _D), lambda i: (0, 0)),
            pl.BlockSpec((_D, 8), lambda i: (0, 0)),
        ],
        out_specs=[
            pl.BlockSpec((_BN, _ROW), lambda i: (i, 0)),
            pl.BlockSpec((8, _BN), lambda i: (0, i)),
        ],
        out_shape=[
            jax.ShapeDtypeStruct((_NPAD, _ROW), jnp.float32),
            jax.ShapeDtypeStruct((8, _NPAD), jnp.float32),
        ],
    )(parts, parts, haug1, am1, b1, w2, am2)


def _final_body(p0_ref, p1_ref, hg_ref, am2_ref, b2_ref, w1_ref, wb1_ref,
                w2_ref, wb2_ref, out_ref):
    hg = hg_ref[...]
    h = hg[:, :_D]
    a = jnp.dot(h, am2_ref[...], preferred_element_type=jnp.float32)
    es = a[:, 0:1] + a[:, 1:2]
    es = jnp.where(es >= 0.0, es, 0.2 * es)
    tot = p0_ref[0] + p1_ref[0] + jnp.exp(es) * hg
    g = jnp.maximum(tot[:, :_D] / tot[:, _D:_D + 1] + b2_ref[...], 0.0)
    z = jnp.maximum(
        jnp.dot(g, w1_ref[...], preferred_element_type=jnp.float32)
        + wb1_ref[...], 0.0)
    logits = jnp.dot(z, w2_ref[...], preferred_element_type=jnp.float32)
    logits = logits + wb2_ref[...]
    m = jnp.max(logits, axis=1, keepdims=True)
    p = jnp.exp(logits - m)
    out_ref[...] = p / jnp.sum(p, axis=1, keepdims=True)


def _final(parts, haug2, am2, b2, fc1_w, fc1_b, fc2_w, fc2_b):
    return pl.pallas_call(
        _final_body,
        grid=(_NPAD // _BN,),
        in_specs=[
            pl.BlockSpec((1, _BN, _ROW), lambda i: (0, i, 0)),
            pl.BlockSpec((1, _BN, _ROW), lambda i: (1, i, 0)),
            pl.BlockSpec((_BN, _ROW), lambda i: (i, 0)),
            pl.BlockSpec((_D, 8), lambda i: (0, 0)),
            pl.BlockSpec((1, _D), lambda i: (0, 0)),
            pl.BlockSpec((_D, _D), lambda i: (0, 0)),
            pl.BlockSpec((1, _D), lambda i: (0, 0)),
            pl.BlockSpec((_D, _OUT), lambda i: (0, 0)),
            pl.BlockSpec((1, _OUT), lambda i: (0, 0)),
        ],
        out_specs=pl.BlockSpec((_BN, _OUT), lambda i: (i, 0)),
        out_shape=jax.ShapeDtypeStruct((_NPAD, _OUT), jnp.float32),
    )(parts, parts, haug2, am2, b2, fc1_w, fc1_b, fc2_w, fc2_b)


def _edge_body(src_hbm, dst_hbm, av_hbm, haug_hbm, out_hbm,
               asrc_v, adst_v, srci, dsti, rows, accum):
    c = lax.axis_index("c")
    s = lax.axis_index("s")
    wid = c * _NS + s

    @pl.loop(0, _CH)
    def _zero(r):
        for k in range(_ROW // 16):
            rows[r, pl.ds(k * 16, 16)] = jnp.zeros((16,), jnp.float32)

    for t in range(_RPT // _CH):
        pltpu.sync_copy(rows, accum.at[pl.ds(s * _RPT + t * _CH, _CH)])
    rem = _RPT % _CH
    if rem:
        pltpu.sync_copy(rows.at[pl.ds(0, rem)],
                        accum.at[pl.ds(s * _RPT + _RPT - rem, rem)])

    pltpu.sync_copy(av_hbm.at[0], asrc_v)
    pltpu.sync_copy(av_hbm.at[1], adst_v)
    plsc.subcore_barrier()

    base_w = wid * _EPW

    @pl.loop(0, _NCHUNK)
    def _chunk(g):
        base = base_w + g * _CH
        pltpu.sync_copy(src_hbm.at[pl.ds(base, _CH)], srci)
        pltpu.sync_copy(dst_hbm.at[pl.ds(base, _CH)], dsti)
        pltpu.sync_copy(haug_hbm.at[srci], rows)

        @pl.loop(0, _CH, step=16)
        def _w(j):
            a_s = plsc.load_gather(asrc_v, [srci[pl.ds(j, 16)]])
            a_d = plsc.load_gather(adst_v, [dsti[pl.ds(j, 16)]])
            e = a_s + a_d
            e = jnp.where(e >= 0.0, e, 0.2 * e)
            ee = jnp.exp(e)
            for t in range(16):
                v = jnp.full((16,), ee[t], jnp.float32)
                for k in range(_ROW // 16):
                    rows[j + t, pl.ds(k * 16, 16)] = (
                        rows[j + t, pl.ds(k * 16, 16)] * v)

        pltpu.sync_copy(rows, accum.at[dsti], add=True)

    plsc.subcore_barrier()
    pltpu.sync_copy(accum.at[pl.ds(s * _RPT, _RPT)],
                    out_hbm.at[c, pl.ds(s * _RPT, _RPT)])


def _edge_pass(src, dst, av, haug):
    mesh = plsc.VectorSubcoreMesh(core_axis_name="c", subcore_axis_name="s",
                                  num_cores=_NC, num_subcores=_NS)
    cp = pltpu.CompilerParams()
    if "needs_layout_passes" in pltpu.CompilerParams.__dataclass_fields__:
        cp = dataclasses.replace(cp, needs_layout_passes=False,
                                 use_tc_tiling_on_sc=False)
    f = pl.kernel(
        _edge_body,
        out_type=jax.ShapeDtypeStruct((_NC, _NPAD, _ROW), jnp.float32),
        mesh=mesh,
        scratch_types=[
            pltpu.VMEM((_NPAD,), jnp.float32),
            pltpu.VMEM((_NPAD,), jnp.float32),
            pltpu.VMEM((_CH,), jnp.int32),
            pltpu.VMEM((_CH,), jnp.int32),
            pltpu.VMEM((_CH, _ROW), jnp.float32),
            pltpu.VMEM_SHARED((_ACC, _ROW), jnp.float32),
        ],
        compiler_params=cp,
    )
    return f(src, dst, av, haug)


def kernel(x, edge_index, W1, att_s1, att_d1, b1, W2, att_s2, att_d2, b2,
           fc1_w, fc1_b, fc2_w, fc2_b):
    xp = jnp.zeros((_NPAD, _D), jnp.float32).at[:_N].set(x)
    src = jnp.full((_EPAD,), _N, jnp.int32).at[:_E].set(edge_index[0])
    dst = jnp.full((_EPAD,), _N, jnp.int32).at[:_E].set(edge_index[1])
    am1 = (jnp.zeros((_D, 8), jnp.float32)
           .at[:, 0].set(att_s1[0]).at[:, 1].set(att_d1[0]))
    am2 = (jnp.zeros((_D, 8), jnp.float32)
           .at[:, 0].set(att_s2[0]).at[:, 1].set(att_d2[0]))

    haug1, av1 = _embed(xp, W1, am1)
    parts1 = _edge_pass(src, dst, av1, haug1)
    haug2, av2 = _combine(parts1, haug1, am1, b1.reshape(1, _D), W2, am2)
    parts2 = _edge_pass(src, dst, av2, haug2)
    out = _final(parts2, haug2, am2, b2.reshape(1, _D),
                 fc1_w, fc1_b.reshape(1, _D), fc2_w, fc2_b.reshape(1, _OUT))
    return out[:_N]
